# baseline (speedup 1.0000x reference)
import functools
import math

import jax
import jax.numpy as jnp
from jax import lax
from jax.experimental import pallas as pl
from jax.experimental.pallas import tpu as pltpu
from jax.experimental.pallas import tpu_sc as plsc

_NUM_CORES = 2
_NUM_SUBCORES = 16
_NUM_WORKERS = _NUM_CORES * _NUM_SUBCORES
_LANES = 16


@functools.lru_cache(maxsize=None)
def _make_lookup(V, D, B, chunk):
    assert B % _NUM_WORKERS == 0
    b_per_w = B // _NUM_WORKERS
    assert b_per_w % chunk == 0
    n_chunks = b_per_w // chunk
    scale = float(math.sqrt(D))
    mesh = plsc.VectorSubcoreMesh(core_axis_name="c", subcore_axis_name="s")

    @functools.partial(
        pl.kernel,
        mesh=mesh,
        out_type=jax.ShapeDtypeStruct((B, D), jnp.float32),
        scratch_types=[
            pltpu.VMEM((b_per_w,), jnp.int32),
            pltpu.VMEM((chunk, D), jnp.float32),
            pltpu.SemaphoreType.DMA,
        ],
        compiler_params=pltpu.CompilerParams(use_tc_tiling_on_sc=False),
    )
    def lookup(table_hbm, idx_hbm, out_hbm, idx_v, rows_v, sem):
        wid = lax.axis_index("s") * _NUM_CORES + lax.axis_index("c")
        base = wid * b_per_w
        pltpu.sync_copy(idx_hbm.at[pl.ds(base, b_per_w)], idx_v)

        def chunk_body(g, carry):
            pltpu.async_copy(
                table_hbm.at[idx_v.at[pl.ds(g * chunk, chunk)]], rows_v, sem
            ).wait()

            def scale_body(r, c2):
                for c in range(D // _LANES):
                    sl = pl.ds(c * _LANES, _LANES)
                    rows_v[r, sl] = rows_v[r, sl] * scale
                return c2

            lax.fori_loop(0, chunk, scale_body, 0)
            pltpu.sync_copy(rows_v, out_hbm.at[pl.ds(base + g * chunk, chunk)])
            return carry

        lax.fori_loop(0, n_chunks, chunk_body, 0)

    return lookup


def kernel(word_indices, embedding_matrix):
    B0, S = word_indices.shape
    V, D = embedding_matrix.shape
    B = B0 * S
    idx = word_indices.reshape(B).astype(jnp.int32)
    lookup = _make_lookup(V, D, B, 128)
    out = lookup(embedding_matrix, idx)
    return out.reshape(B0, S, D)
